# baseline (speedup 1.0000x reference)
import functools

import jax
import jax.numpy as jnp
from jax import lax
from jax.experimental import pallas as pl
from jax.experimental.pallas import tpu as pltpu
from jax.experimental.pallas import tpu_sc as plsc

_VOCAB = 1000000
_EMBED = 32
_BATCH = 16384
_C = 5
_PACK = 8
_W = _EMBED // 2

_NC = 2
_NS = 16
_NW = _NC * _NS
_BW = _BATCH // _NW
_CW = _BW * _C
_CHUNK = 128
_NJ = _BW // _CHUNK


def _body(tgt_hbm, ctx_hbm, ttab_hbm, ctab_hbm, out_hbm,
          t_stage, c_stage, idx_t, q_t, idx_c, q_c,
          t_chunk, c_chunk0, c_chunk1, out_stage, sem):
  wid = lax.axis_index("s") * _NC + lax.axis_index("c")
  base = wid * _BW

  lanes = lax.iota(jnp.int32, 16)
  zeros16 = jnp.zeros((16,), jnp.int32)

  for s in range(_NJ):
    pltpu.sync_copy(tgt_hbm.at[pl.ds(base + s * _CHUNK, _CHUNK)], t_stage)
    pltpu.sync_copy(ctx_hbm.at[pl.ds(base + s * _CHUNK, _CHUNK)], c_stage)
    for i in range(_CHUNK // 16):
      b16 = i * 16 + lanes
      o = s * _CHUNK + i * 16
      tv = plsc.load_gather(t_stage, [b16, zeros16])
      idx_t[pl.ds(o, 16)] = lax.shift_right_logical(tv, 3)
      q_t[pl.ds(o, 16)] = lax.bitwise_and(tv, _PACK - 1) * _W
      for c in range(_C):
        cv = plsc.load_gather(c_stage, [b16, jnp.full((16,), c, jnp.int32)])
        idx_c[pl.ds(c * _BW + o, 16)] = lax.shift_right_logical(cv, 3)
        q_c[pl.ds(c * _BW + o, 16)] = lax.bitwise_and(cv, _PACK - 1) * _W

  c_bufs = (c_chunk0, c_chunk1)

  def fire_t(j):
    return pltpu.async_copy(
        ttab_hbm.at[idx_t.at[pl.ds(j * _CHUNK, _CHUNK)]], t_chunk, sem)

  def fire_c(j, c):
    return pltpu.async_copy(
        ctab_hbm.at[idx_c.at[pl.ds(c * _BW + j * _CHUNK, _CHUNK)]],
        c_bufs[c % 2], sem)

  for j in range(_NJ):
    fire_t(j).wait()
    cp = fire_c(j, 0)
    for c in range(_C):
      cp.wait()
      if c + 1 < _C:
        cp = fire_c(j, c + 1)
      buf = c_bufs[c % 2]

      def grp(i, _, c=c, j=j, buf=buf):
        b16 = i * 16 + lanes
        tq = q_t[pl.ds(j * _CHUNK + i * 16, 16)]
        cq = q_c[pl.ds(c * _BW + j * _CHUNK + i * 16, 16)]
        acc = jnp.zeros((16,), jnp.float32)
        for e2 in range(_W):
          wp = plsc.load_gather(t_chunk, [b16, tq + e2])
          xp = plsc.load_gather(buf, [b16, cq + e2])
          wb = plsc.bitcast(wp, jnp.bfloat16)
          xb = plsc.bitcast(xp, jnp.bfloat16)
          w0, w1 = plsc.unpack(wb, format=plsc.PackFormat.INTERLEAVED)
          x0, x1 = plsc.unpack(xb, format=plsc.PackFormat.INTERLEAVED)
          acc = acc + w0 * x0 + w1 * x1
        plsc.store_scatter(out_stage, [b16, jnp.full((16,), c, jnp.int32)],
                           acc)
        return ()

      lax.fori_loop(0, _CHUNK // 16, grp, ())

    pltpu.sync_copy(out_stage, out_hbm.at[pl.ds(base + j * _CHUNK, _CHUNK)])


@jax.jit
def _run(target, context, ttab_p, ctab_p):
  mesh = plsc.VectorSubcoreMesh(core_axis_name="c", subcore_axis_name="s")
  k = functools.partial(
      pl.kernel,
      mesh=mesh,
      compiler_params=pltpu.CompilerParams(needs_layout_passes=False),
      out_type=jax.ShapeDtypeStruct((_BATCH, _C), jnp.float32),
      scratch_types=[
          pltpu.VMEM((_CHUNK, 1), jnp.int32),
          pltpu.VMEM((_CHUNK, _C), jnp.int32),
          pltpu.VMEM((_BW,), jnp.int32),
          pltpu.VMEM((_BW,), jnp.int32),
          pltpu.VMEM((_CW,), jnp.int32),
          pltpu.VMEM((_CW,), jnp.int32),
          pltpu.VMEM((_CHUNK, 128), jnp.int32),
          pltpu.VMEM((_CHUNK, 128), jnp.int32),
          pltpu.VMEM((_CHUNK, 128), jnp.int32),
          pltpu.VMEM((_CHUNK, _C), jnp.float32),
          pltpu.SemaphoreType.DMA,
      ],
  )(_body)
  return k(target, context, ttab_p, ctab_p)


def _pack_table(tab):
  u = lax.bitcast_convert_type(tab, jnp.uint32)
  bf = lax.shift_right_logical(u + jnp.uint32(0x8000), jnp.uint32(16))
  lo = bf[:, 0::2]
  hi = bf[:, 1::2]
  packed = lax.bitwise_or(lo, lax.shift_left(hi, jnp.uint32(16)))
  i32 = lax.bitcast_convert_type(packed, jnp.int32)
  return i32.reshape(_VOCAB // _PACK, _PACK * _W)


def kernel(target, context, target_table, context_table):
  return _run(target, context,
              _pack_table(target_table), _pack_table(context_table))


# speedup vs baseline: 2.5300x; 2.5300x over previous
import functools

import jax
import jax.numpy as jnp
from jax import lax
from jax.experimental import pallas as pl
from jax.experimental.pallas import tpu as pltpu
from jax.experimental.pallas import tpu_sc as plsc

_VOCAB = 1000000
_EMBED = 32
_BATCH = 16384
_C = 5

_NC = 2
_NS = 16
_NW = _NC * _NS
_BW = _BATCH // _NW
_CW = _BW * _C
_CHUNK = 128
_GB = 16


def _body(tgt_hbm, ctx_hbm, ttab_hbm, ctab_hbm, out_hbm,
          idx_t, idx_c, rows_t, rows_c, out_v, sem):
  wid = lax.axis_index("s") * _NC + lax.axis_index("c")

  pltpu.sync_copy(tgt_hbm.at[wid], idx_t)
  pltpu.sync_copy(ctx_hbm.at[wid], idx_c)

  copies = []
  for j in range(_BW // _CHUNK):
    copies.append(pltpu.async_copy(
        ttab_hbm.at[idx_t.at[j]],
        rows_t.at[pl.ds(j * _CHUNK, _CHUNK)], sem))
  for j in range(_CW // _CHUNK):
    copies.append(pltpu.async_copy(
        ctab_hbm.at[idx_c.at[j]],
        rows_c.at[pl.ds(j * _CHUNK, _CHUNK)], sem))
  for cp in copies:
    cp.wait()

  lanes = lax.iota(jnp.int32, 16)
  masks = [lanes == l for l in range(16)]

  def step(i, _):
    accs = [jnp.zeros((16,), jnp.float32) for _ in range(_C)]
    for k in range(_GB):
      b = i * _GB + k
      w0 = rows_t[b, pl.ds(0, 16)]
      w1 = rows_t[b, pl.ds(16, 16)]
      for c in range(_C):
        r = b * _C + c
        p = w0 * rows_c[r, pl.ds(0, 16)] + w1 * rows_c[r, pl.ds(16, 16)]
        s = jnp.sum(p)
        q = k * _C + c
        accs[q // 16] = jnp.where(masks[q % 16], s, accs[q // 16])
    base = i * (_GB * _C)
    for j in range(_C):
      out_v[pl.ds(base + j * 16, 16)] = accs[j]
    return ()

  lax.fori_loop(0, _BW // _GB, step, ())

  pltpu.sync_copy(out_v, out_hbm.at[wid])


@jax.jit
def _run(tgt3d, ctx3d, target_table, context_table):
  mesh = plsc.VectorSubcoreMesh(core_axis_name="c", subcore_axis_name="s")
  k = functools.partial(
      pl.kernel,
      mesh=mesh,
      compiler_params=pltpu.CompilerParams(
          use_tc_tiling_on_sc=False, needs_layout_passes=False),
      out_type=jax.ShapeDtypeStruct((_NW, _CW), jnp.float32),
      scratch_types=[
          pltpu.VMEM((_BW // _CHUNK, _CHUNK), jnp.int32),
          pltpu.VMEM((_CW // _CHUNK, _CHUNK), jnp.int32),
          pltpu.VMEM((_BW, _EMBED), jnp.float32),
          pltpu.VMEM((_CW, _EMBED), jnp.float32),
          pltpu.VMEM((_CW,), jnp.float32),
          pltpu.SemaphoreType.DMA,
      ],
  )(_body)
  return k(tgt3d, ctx3d, target_table, context_table)


def kernel(target, context, target_table, context_table):
  tgt3d = target.reshape(_NW, _BW // _CHUNK, _CHUNK)
  ctx3d = context.reshape(_NW, _CW // _CHUNK, _CHUNK)
  out = _run(tgt3d, ctx3d, target_table, context_table)
  return out.reshape(_BATCH, _C)
